# baseline (speedup 1.0000x reference)
import dataclasses

import jax
import jax.numpy as jnp
from jax import lax
from jax.experimental import pallas as pl
from jax.experimental.pallas import tpu as pltpu
from jax.experimental.pallas import tpu_sc as plsc

ROWS = 8192
DIM = 4096

NC = 2
NS = 16
NW = NC * NS
R_PER_W = ROWS // NW
CH = 4
HALF = CH // 2
NCH = R_PER_W // CH
NGRP = DIM // 16
UNROLL = 8
NBUF = 4


def _shuffle2(perm_v, in_b, out_b, lr0):
    rvecs = [jnp.full((16,), lr0 + rr, jnp.int32) for rr in range(HALF)]

    @pl.loop(0, NGRP, step=UNROLL)
    def _(j):
        base = j * 16
        idxs = [perm_v[pl.ds(base + u * 16, 16)] for u in range(UNROLL)]
        vals = [
            plsc.load_gather(in_b, [rvecs[rr], idxs[u]])
            for rr in range(HALF)
            for u in range(UNROLL)
        ]
        k = 0
        for rr in range(HALF):
            for u in range(UNROLL):
                out_b[rr, pl.ds(base + u * 16, 16)] = vals[k]
                k += 1


def _sc_body(x_hbm, perm_hbm, o_hbm, perm_v,
             in0, in1, in2, in3, oa0, oa1, oa2, oa3,
             si0, si1, si2, si3, so0, so1, so2, so3):
    ins = [in0, in1, in2, in3]
    outs = [oa0, oa1, oa2, oa3]
    sis = [si0, si1, si2, si3]
    sos = [so0, so1, so2, so3]

    wid = lax.axis_index("s") * NC + lax.axis_index("c")
    base = wid * R_PER_W

    pltpu.sync_copy(perm_hbm, perm_v)

    def chunk(c):
        return pl.ds(base + c * CH, CH)

    def half(c, h):
        return pl.ds(base + c * CH + h * HALF, HALF)

    for k in range(NBUF):
        pltpu.async_copy(x_hbm.at[chunk(k)], ins[k], sis[k])

    @pl.loop(0, NCH, step=NBUF)
    def _(c):
        for k in range(NBUF):
            cc = c + k
            in_b, si = ins[k], sis[k]
            pltpu.make_async_copy(x_hbm.at[chunk(cc)], in_b, si).wait()

            for h in range(2):
                ob_i = 2 * (k % 2) + h
                ob, so = outs[ob_i], sos[ob_i]

                @pl.when(cc >= 2)
                def _(ob=ob, so=so, cc=cc, h=h):
                    pltpu.make_async_copy(
                        ob, o_hbm.at[half(cc - 2, h)], so
                    ).wait()

                _shuffle2(perm_v, in_b, ob, h * HALF)
                pltpu.async_copy(ob, o_hbm.at[half(cc, h)], so)

            @pl.when(cc + NBUF < NCH)
            def _(in_b=in_b, si=si, cc=cc):
                pltpu.async_copy(x_hbm.at[chunk(cc + NBUF)], in_b, si)

    for cc in (NCH - 2, NCH - 1):
        for h in range(2):
            ob_i = 2 * (cc % 2) + h
            pltpu.make_async_copy(
                outs[ob_i], o_hbm.at[half(cc, h)], sos[ob_i]
            ).wait()


def kernel(x, perm):
    mesh = plsc.VectorSubcoreMesh(core_axis_name="c", subcore_axis_name="s")
    cp = pltpu.CompilerParams()
    if "needs_layout_passes" in pltpu.CompilerParams.__dataclass_fields__:
        cp = dataclasses.replace(cp, needs_layout_passes=False)
    kfn = pl.kernel(
        _sc_body,
        mesh=mesh,
        compiler_params=cp,
        out_type=jax.ShapeDtypeStruct((ROWS, DIM), jnp.float32),
        scratch_types=(
            [pltpu.VMEM((DIM,), jnp.int32)]
            + [pltpu.VMEM((CH, DIM), jnp.float32) for _ in range(4)]
            + [pltpu.VMEM((HALF, DIM), jnp.float32) for _ in range(4)]
            + [pltpu.SemaphoreType.DMA for _ in range(8)]
        ),
    )
    return kfn(x, perm)
